# Initial kernel scaffold; baseline (speedup 1.0000x reference)
import jax
import jax.numpy as jnp
from jax.experimental import pallas as pl


def kernel(x, edge_index, batch, embed, Wrel0, brel0, Wroot0, Wrel, brel, Wroot, Wlin, blin, Wout, bout):
    raise NotImplementedError("write your pallas kernel here")


# speedup vs baseline: 1.6072x; 1.6072x over previous
import functools

import jax
import jax.numpy as jnp
from jax import lax
from jax.experimental import pallas as pl
from jax.experimental.pallas import tpu as pltpu
from jax.experimental.pallas import tpu_sc as plsc

N = 10000
E = 320000
NG = 256
NP = 10240
K = 128
NCH = 160
EPT = NCH * K
EPAD = 16 * EPT
HH = 5120
HHP = 5248
NGP = 272

@functools.lru_cache(maxsize=None)
def _mesh():
    return plsc.VectorSubcoreMesh(core_axis_name="c", subcore_axis_name="s",
                                  num_cores=2, num_subcores=16)


def _leaky(v):
    return jnp.where(v > 0, v, 0.01 * v)


def _fill(ref, rows, value):
    width = ref.shape[1]
    vec = jnp.full((16,), value, jnp.float32)

    def body(i, _):
        for j in range(width // 16):
            ref[i, pl.ds(j * 16, 16)] = vec
        return 0

    lax.fori_loop(0, rows, body, 0)


@functools.lru_cache(maxsize=None)
def _make_agg(D):
    @functools.partial(
        pl.kernel,
        out_type=jax.ShapeDtypeStruct((NP, D), jnp.float32),
        mesh=_mesh(),
        scratch_types=[
            pltpu.VMEM((NCH, K), jnp.int32),
            pltpu.VMEM((NCH, K), jnp.int32),
            pltpu.VMEM((K, D), jnp.float32),
            pltpu.VMEM((K, D), jnp.float32),
            pltpu.VMEM_SHARED((HHP, D), jnp.float32),
            pltpu.SemaphoreType.DMA,
        ],
    )
    def agg_kernel(src3d, dst3d, h, out, srcv, dstv, rows, zbuf, aggS, sem):
        c = lax.axis_index("c")
        s = lax.axis_index("s")

        _fill(zbuf, K, 0.0)
        zb = s * (HHP // 16)
        pltpu.sync_copy(zbuf, aggS.at[pl.ds(zb, K)])
        pltpu.sync_copy(zbuf, aggS.at[pl.ds(zb + K, K)])
        pltpu.sync_copy(zbuf.at[pl.ds(0, 72)], aggS.at[pl.ds(zb + 2 * K, 72)])

        pltpu.sync_copy(src3d.at[s], srcv)
        pltpu.sync_copy(dst3d.at[s], dstv)

        base = c * HH

        def xform(j, _):
            for k in range(K // 16):
                d = dstv[j, pl.ds(k * 16, 16)] - base
                m = (d >= 0) & (d < HH)
                dstv[j, pl.ds(k * 16, 16)] = jnp.where(m, d, HH)
            return 0

        lax.fori_loop(0, NCH, xform, 0)
        plsc.subcore_barrier()

        def body(j, _):
            pltpu.async_copy(h.at[srcv.at[j]], rows, sem).wait()
            pltpu.sync_copy(rows, aggS.at[dstv.at[j]], add=True)
            return 0

        lax.fori_loop(0, NCH, body, 0)
        plsc.subcore_barrier()

        for off, sz in ((0, K), (K, K), (2 * K, 64)):
            pltpu.sync_copy(aggS.at[pl.ds(s * 320 + off, sz)],
                            rows.at[pl.ds(0, sz)])
            pltpu.sync_copy(rows.at[pl.ds(0, sz)],
                            out.at[pl.ds(base + s * 320 + off, sz)])

    return agg_kernel


@functools.lru_cache(maxsize=None)
def _make_embed_gather():
    @functools.partial(
        pl.kernel,
        out_type=jax.ShapeDtypeStruct((NP, 128), jnp.float32),
        mesh=_mesh(),
        scratch_types=[
            pltpu.VMEM((4, 80), jnp.int32),
            pltpu.VMEM((80, 128), jnp.float32),
            pltpu.SemaphoreType.DMA,
        ],
    )
    def _embed_gather(x3d, embed2, out, idxv, rows, sem):
        c = lax.axis_index("c")
        s = lax.axis_index("s")
        w = s * 2 + c
        pltpu.sync_copy(x3d.at[w], idxv)
        for j in range(4):
            pltpu.async_copy(embed2.at[idxv.at[j]], rows, sem).wait()
            pltpu.sync_copy(rows, out.at[pl.ds(w * 320 + j * 80, 80)])

    return _embed_gather


def _fixup_body(hw, par, out):
    a = hw[...]
    sel = jnp.where(par[...] > 0, a[:, 64:128], a[:, 0:64])
    out[...] = jnp.concatenate([sel, jnp.zeros_like(sel)], axis=1)


def _tc_fixup(hw, par):
    return pl.pallas_call(
        _fixup_body,
        out_shape=jax.ShapeDtypeStruct((NP, 128), jnp.float32),
        grid=(NP // _BR,),
        in_specs=[pl.BlockSpec((_BR, 128), lambda i: (i, 0)),
                  pl.BlockSpec((_BR, 1), lambda i: (i, 0))],
        out_specs=pl.BlockSpec((_BR, 128), lambda i: (i, 0)),
    )(hw, par)


@functools.lru_cache(maxsize=None)
def _make_pool():
    @functools.partial(
        pl.kernel,
        out_type=(
            jax.ShapeDtypeStruct((2, NGP, 128), jnp.float32),
            jax.ShapeDtypeStruct((2, NGP, 128), jnp.float32),
            jax.ShapeDtypeStruct((2, NGP, 128), jnp.float32),
        ),
        mesh=_mesh(),
        scratch_types=[
            pltpu.VMEM((4, 80), jnp.int32),
            pltpu.VMEM((80, 128), jnp.float32),
            pltpu.VMEM((80, 128), jnp.float32),
            pltpu.VMEM((128, 128), jnp.float32),
            pltpu.VMEM_SHARED((NGP, 128), jnp.float32),
            pltpu.VMEM_SHARED((NGP, 128), jnp.float32),
            pltpu.VMEM_SHARED((NGP, 128), jnp.float32),
            pltpu.SemaphoreType.DMA,
        ],
    )
    def _pool(batch3d, xh, y, px, py, pc, idxv, rowsx, rowsy, zbuf,
              pxS, pyS, pcS, sem):
        c = lax.axis_index("c")
        s = lax.axis_index("s")
        w = s * 2 + c

        _fill(zbuf, 128, 0.0)

        @pl.when(s == 0)
        def _():
            for ref in (pxS, pyS, pcS):
                pltpu.sync_copy(zbuf, ref.at[pl.ds(0, 128)])
                pltpu.sync_copy(zbuf, ref.at[pl.ds(128, 128)])
                pltpu.sync_copy(zbuf.at[pl.ds(0, 16)], ref.at[pl.ds(256, 16)])

        pltpu.sync_copy(batch3d.at[w], idxv)
        _fill(rowsy, 80, 1.0)
        plsc.subcore_barrier()

        for j in range(4):
            pltpu.sync_copy(rowsy, pcS.at[idxv.at[j]], add=True)
        for j in range(4):
            pltpu.sync_copy(xh.at[pl.ds(w * 320 + j * 80, 80)], rowsx)
            pltpu.sync_copy(rowsx, pxS.at[idxv.at[j]], add=True)
            pltpu.sync_copy(y.at[pl.ds(w * 320 + j * 80, 80)], rowsy)
            pltpu.sync_copy(rowsy, pyS.at[idxv.at[j]], add=True)

        plsc.subcore_barrier()

        @pl.when(s == 0)
        def _():
            for ref, out in ((pxS, px), (pyS, py), (pcS, pc)):
                pltpu.sync_copy(ref.at[pl.ds(0, 128)], zbuf)
                pltpu.sync_copy(zbuf, out.at[c, pl.ds(0, 128)])
                pltpu.sync_copy(ref.at[pl.ds(128, 128)], zbuf)
                pltpu.sync_copy(zbuf, out.at[c, pl.ds(128, 128)])
                pltpu.sync_copy(ref.at[pl.ds(256, 16)], zbuf.at[pl.ds(0, 16)])
                pltpu.sync_copy(zbuf.at[pl.ds(0, 16)], out.at[c, pl.ds(256, 16)])

    return _pool


_BR = 256


def _layer_body(agg, h, Wr, br, Wt, out):
    r = (jnp.dot(agg[...], Wr[...], preferred_element_type=jnp.float32) + br[...]
         + jnp.dot(h[...], Wt[...], preferred_element_type=jnp.float32))
    out[...] = _leaky(r)


def _layer_sum_body(agg, h, other, Wr, br, Wt, out, outsum):
    r = (jnp.dot(agg[...], Wr[...], preferred_element_type=jnp.float32) + br[...]
         + jnp.dot(h[...], Wt[...], preferred_element_type=jnp.float32))
    r = _leaky(r)
    out[...] = r
    outsum[...] = r + other[...]


def _row_spec(D):
    return pl.BlockSpec((_BR, D), lambda i: (i, 0))


def _full_spec(shape):
    return pl.BlockSpec(shape, lambda i: tuple(0 for _ in shape))


def _tc_layer(agg, h, Wr, br, Wt):
    D = h.shape[1]
    return pl.pallas_call(
        _layer_body,
        out_shape=jax.ShapeDtypeStruct((NP, 128), jnp.float32),
        grid=(NP // _BR,),
        in_specs=[_row_spec(D), _row_spec(D),
                  _full_spec((D, 128)), _full_spec((1, 128)),
                  _full_spec((D, 128))],
        out_specs=_row_spec(128),
    )(agg, h, Wr, br.reshape(1, 128), Wt)


def _tc_layer_sum(agg, h, other, Wr, br, Wt):
    D = h.shape[1]
    return pl.pallas_call(
        _layer_sum_body,
        out_shape=(jax.ShapeDtypeStruct((NP, 128), jnp.float32),
                   jax.ShapeDtypeStruct((NP, 128), jnp.float32)),
        grid=(NP // _BR,),
        in_specs=[_row_spec(D), _row_spec(D), _row_spec(128),
                  _full_spec((D, 128)), _full_spec((1, 128)),
                  _full_spec((D, 128))],
        out_specs=(_row_spec(128), _row_spec(128)),
    )(agg, h, other, Wr, br.reshape(1, 128), Wt)


def _mlp_body(px, py, pc, Wlin, blin, Wout, bout, out):
    cnt = jnp.maximum(pc[0, :NG] + pc[1, :NG], 1.0)
    mx = (px[0, :NG] + px[1, :NG]) / cnt
    my = (py[0, :NG] + py[1, :NG]) / cnt
    z = jnp.concatenate([mx, my], axis=1)

    def lin(z, i):
        z = _leaky(jnp.dot(z, Wlin[2 * i], preferred_element_type=jnp.float32)
                   + blin[2 * i])
        z = _leaky(jnp.dot(z, Wlin[2 * i + 1], preferred_element_type=jnp.float32)
                   + blin[2 * i + 1])
        return z

    zy = lin(z, 0)
    z = lin(z + zy, 1)
    zy = lin(z + zy, 2)
    z = lin(z + zy, 3)
    out[...] = jnp.dot(z, Wout[...], preferred_element_type=jnp.float32) + bout[...]


def _tc_mlp(px, py, pc, Wlin, blin, Wout, bout):
    Woutp = jnp.zeros((256, 128), jnp.float32).at[:, 0:1].set(Wout)
    boutp = jnp.zeros((1, 128), jnp.float32).at[0, 0].set(bout[0])
    return pl.pallas_call(
        _mlp_body,
        out_shape=jax.ShapeDtypeStruct((NG, 128), jnp.float32),
        in_specs=[pl.BlockSpec((2, NGP, 128), lambda: (0, 0, 0)),
                  pl.BlockSpec((2, NGP, 128), lambda: (0, 0, 0)),
                  pl.BlockSpec((2, NGP, 128), lambda: (0, 0, 0)),
                  pl.BlockSpec((8, 256, 256), lambda: (0, 0, 0)),
                  pl.BlockSpec((8, 1, 256), lambda: (0, 0, 0)),
                  pl.BlockSpec((256, 128), lambda: (0, 0)),
                  pl.BlockSpec((1, 128), lambda: (0, 0))],
        out_specs=pl.BlockSpec((NG, 128), lambda: (0, 0)),
    )(px, py, pc, Wlin, blin.reshape(8, 1, 256), Woutp, boutp)


def kernel(x, edge_index, batch, embed, Wrel0, brel0, Wroot0, Wrel, brel,
           Wroot, Wlin, blin, Wout, bout):
    i32 = jnp.int32
    src3d = jnp.pad(edge_index[0].astype(i32), (0, EPAD - E)).reshape(16, NCH, K)
    dst3d = jnp.pad(edge_index[1].astype(i32), (0, EPAD - E),
                    constant_values=N).reshape(16, NCH, K)
    xp = jnp.pad(x.astype(i32), (0, NP - N))
    x3d = (xp // 2).reshape(32, 4, 80)
    xpar = (xp % 2).astype(jnp.float32).reshape(NP, 1)
    batch3d = jnp.pad(batch.astype(i32), (0, NP - N),
                      constant_values=NG).reshape(32, 4, 80)
    embed2 = embed.reshape(embed.shape[0] // 2, 128)
    Wrel0p = jnp.pad(Wrel0, ((0, 64), (0, 0)))
    Wroot0p = jnp.pad(Wroot0, ((0, 64), (0, 0)))

    h = _tc_fixup(_make_embed_gather()(x3d, embed2), xpar)

    def conv(h, Wr, br, Wt):
        agg = _make_agg(128)(src3d, dst3d, h)
        return _tc_layer(agg, h, Wr, br, Wt)

    def conv_sum(h, other, Wr, br, Wt):
        agg = _make_agg(128)(src3d, dst3d, h)
        return _tc_layer_sum(agg, h, other, Wr, br, Wt)

    h = conv(h, Wrel0p, brel0, Wroot0p)
    xh = conv(h, Wrel[0], brel[0], Wroot[0])

    t = conv(xh, Wrel[1], brel[1], Wroot[1])
    y, s = conv_sum(t, xh, Wrel[2], brel[2], Wroot[2])
    t = conv(s, Wrel[3], brel[3], Wroot[3])
    xh, s = conv_sum(t, y, Wrel[4], brel[4], Wroot[4])
    t = conv(s, Wrel[5], brel[5], Wroot[5])
    y, s = conv_sum(t, xh, Wrel[6], brel[6], Wroot[6])
    t = conv(s, Wrel[7], brel[7], Wroot[7])
    xh, s = conv_sum(t, y, Wrel[8], brel[8], Wroot[8])
    t = conv(s, Wrel[9], brel[9], Wroot[9])
    y = conv(t, Wrel[10], brel[10], Wroot[10])

    px, py, pc = _make_pool()(batch3d, xh, y)
    out = _tc_mlp(px, py, pc, Wlin, blin, Wout, bout)
    return out[:, 0:1]


# speedup vs baseline: 1.7054x; 1.0611x over previous
import functools

import jax
import jax.numpy as jnp
from jax import lax
from jax.experimental import pallas as pl
from jax.experimental.pallas import tpu as pltpu
from jax.experimental.pallas import tpu_sc as plsc

N = 10000
E = 320000
NG = 256
NP = 10240
K = 128
NCH = 160
EPT = NCH * K
EPAD = 16 * EPT
HH = 5120
HHP = 5248
NGP = 272

@functools.lru_cache(maxsize=None)
def _mesh():
    return plsc.VectorSubcoreMesh(core_axis_name="c", subcore_axis_name="s",
                                  num_cores=2, num_subcores=16)


def _leaky(v):
    return jnp.where(v > 0, v, 0.01 * v)


def _fill(ref, rows, value):
    width = ref.shape[1]
    vec = jnp.full((16,), value, jnp.float32)

    def body(i, _):
        for j in range(width // 16):
            ref[i, pl.ds(j * 16, 16)] = vec
        return 0

    lax.fori_loop(0, rows, body, 0)


@functools.lru_cache(maxsize=None)
def _make_agg(D):
    @functools.partial(
        pl.kernel,
        out_type=jax.ShapeDtypeStruct((NP, D), jnp.float32),
        mesh=_mesh(),
        scratch_types=[
            pltpu.VMEM((NCH, K), jnp.int32),
            pltpu.VMEM((NCH, K), jnp.int32),
            pltpu.VMEM((2, K, D), jnp.float32),
            pltpu.VMEM_SHARED((HHP, D), jnp.float32),
            pltpu.SemaphoreType.DMA((2,)),
        ],
    )
    def agg_kernel(src3d, dst3d, h, out, srcv, dstv, rows, aggS, semgs):
        c = lax.axis_index("c")
        s = lax.axis_index("s")

        zbuf = rows.at[0]
        _fill(zbuf, K, 0.0)
        zb = s * (HHP // 16)
        pltpu.sync_copy(zbuf, aggS.at[pl.ds(zb, K)])
        pltpu.sync_copy(zbuf, aggS.at[pl.ds(zb + K, K)])
        pltpu.sync_copy(zbuf.at[pl.ds(0, 72)], aggS.at[pl.ds(zb + 2 * K, 72)])

        pltpu.sync_copy(src3d.at[s], srcv)
        pltpu.sync_copy(dst3d.at[s], dstv)

        base = c * HH

        def xform(j, _):
            for k in range(K // 16):
                d = dstv[j, pl.ds(k * 16, 16)] - base
                m = (d >= 0) & (d < HH)
                dstv[j, pl.ds(k * 16, 16)] = jnp.where(m, d, HH)
            return 0

        lax.fori_loop(0, NCH, xform, 0)
        plsc.subcore_barrier()

        def gather(j, b):
            pltpu.async_copy(h.at[srcv.at[j]], rows.at[b], semgs.at[b])

        gather(0, 0)

        def body(jo, _):
            for b in range(2):
                j = jo * 2 + b
                pltpu.make_async_copy(h.at[srcv.at[0]], rows.at[b],
                                      semgs.at[b]).wait()

                @pl.when(j + 1 < NCH)
                def _():
                    gather(j + 1, (b + 1) % 2)

                pltpu.sync_copy(rows.at[b], aggS.at[dstv.at[j]], add=True)
            return 0

        lax.fori_loop(0, NCH // 2, body, 0)
        plsc.subcore_barrier()

        for off, sz in ((0, K), (K, K), (2 * K, 64)):
            pltpu.sync_copy(aggS.at[pl.ds(s * 320 + off, sz)],
                            rows.at[0, pl.ds(0, sz)])
            pltpu.sync_copy(rows.at[0, pl.ds(0, sz)],
                            out.at[pl.ds(base + s * 320 + off, sz)])

    return agg_kernel


@functools.lru_cache(maxsize=None)
def _make_embed_gather():
    @functools.partial(
        pl.kernel,
        out_type=jax.ShapeDtypeStruct((NP, 128), jnp.float32),
        mesh=_mesh(),
        scratch_types=[
            pltpu.VMEM((4, 80), jnp.int32),
            pltpu.VMEM((80, 128), jnp.float32),
            pltpu.SemaphoreType.DMA,
        ],
    )
    def _embed_gather(x3d, embed2, out, idxv, rows, sem):
        c = lax.axis_index("c")
        s = lax.axis_index("s")
        w = s * 2 + c
        pltpu.sync_copy(x3d.at[w], idxv)
        for j in range(4):
            pltpu.async_copy(embed2.at[idxv.at[j]], rows, sem).wait()
            pltpu.sync_copy(rows, out.at[pl.ds(w * 320 + j * 80, 80)])

    return _embed_gather


def _fixup_body(hw, par, out):
    a = hw[...]
    sel = jnp.where(par[...] > 0, a[:, 64:128], a[:, 0:64])
    out[...] = jnp.concatenate([sel, jnp.zeros_like(sel)], axis=1)


def _tc_fixup(hw, par):
    return pl.pallas_call(
        _fixup_body,
        out_shape=jax.ShapeDtypeStruct((NP, 128), jnp.float32),
        grid=(NP // _BR,),
        in_specs=[pl.BlockSpec((_BR, 128), lambda i: (i, 0)),
                  pl.BlockSpec((_BR, 1), lambda i: (i, 0))],
        out_specs=pl.BlockSpec((_BR, 128), lambda i: (i, 0)),
    )(hw, par)


@functools.lru_cache(maxsize=None)
def _make_pool():
    @functools.partial(
        pl.kernel,
        out_type=(
            jax.ShapeDtypeStruct((2, NGP, 128), jnp.float32),
            jax.ShapeDtypeStruct((2, NGP, 128), jnp.float32),
            jax.ShapeDtypeStruct((2, NGP, 128), jnp.float32),
        ),
        mesh=_mesh(),
        scratch_types=[
            pltpu.VMEM((4, 80), jnp.int32),
            pltpu.VMEM((80, 128), jnp.float32),
            pltpu.VMEM((80, 128), jnp.float32),
            pltpu.VMEM((128, 128), jnp.float32),
            pltpu.VMEM_SHARED((NGP, 128), jnp.float32),
            pltpu.VMEM_SHARED((NGP, 128), jnp.float32),
            pltpu.VMEM_SHARED((NGP, 128), jnp.float32),
            pltpu.SemaphoreType.DMA,
        ],
    )
    def _pool(batch3d, xh, y, px, py, pc, idxv, rowsx, rowsy, zbuf,
              pxS, pyS, pcS, sem):
        c = lax.axis_index("c")
        s = lax.axis_index("s")
        w = s * 2 + c

        _fill(zbuf, 128, 0.0)

        @pl.when(s == 0)
        def _():
            for ref in (pxS, pyS, pcS):
                pltpu.sync_copy(zbuf, ref.at[pl.ds(0, 128)])
                pltpu.sync_copy(zbuf, ref.at[pl.ds(128, 128)])
                pltpu.sync_copy(zbuf.at[pl.ds(0, 16)], ref.at[pl.ds(256, 16)])

        pltpu.sync_copy(batch3d.at[w], idxv)
        _fill(rowsy, 80, 1.0)
        plsc.subcore_barrier()

        for j in range(4):
            pltpu.sync_copy(rowsy, pcS.at[idxv.at[j]], add=True)
        for j in range(4):
            pltpu.sync_copy(xh.at[pl.ds(w * 320 + j * 80, 80)], rowsx)
            pltpu.sync_copy(rowsx, pxS.at[idxv.at[j]], add=True)
            pltpu.sync_copy(y.at[pl.ds(w * 320 + j * 80, 80)], rowsy)
            pltpu.sync_copy(rowsy, pyS.at[idxv.at[j]], add=True)

        plsc.subcore_barrier()

        @pl.when(s == 0)
        def _():
            for ref, out in ((pxS, px), (pyS, py), (pcS, pc)):
                pltpu.sync_copy(ref.at[pl.ds(0, 128)], zbuf)
                pltpu.sync_copy(zbuf, out.at[c, pl.ds(0, 128)])
                pltpu.sync_copy(ref.at[pl.ds(128, 128)], zbuf)
                pltpu.sync_copy(zbuf, out.at[c, pl.ds(128, 128)])
                pltpu.sync_copy(ref.at[pl.ds(256, 16)], zbuf.at[pl.ds(0, 16)])
                pltpu.sync_copy(zbuf.at[pl.ds(0, 16)], out.at[c, pl.ds(256, 16)])

    return _pool


_BR = 256


def _layer_body(agg, h, Wr, br, Wt, out):
    r = (jnp.dot(agg[...], Wr[...], preferred_element_type=jnp.float32) + br[...]
         + jnp.dot(h[...], Wt[...], preferred_element_type=jnp.float32))
    out[...] = _leaky(r)


def _layer_sum_body(agg, h, other, Wr, br, Wt, out, outsum):
    r = (jnp.dot(agg[...], Wr[...], preferred_element_type=jnp.float32) + br[...]
         + jnp.dot(h[...], Wt[...], preferred_element_type=jnp.float32))
    r = _leaky(r)
    out[...] = r
    outsum[...] = r + other[...]


def _row_spec(D):
    return pl.BlockSpec((_BR, D), lambda i: (i, 0))


def _full_spec(shape):
    return pl.BlockSpec(shape, lambda i: tuple(0 for _ in shape))


def _tc_layer(agg, h, Wr, br, Wt):
    D = h.shape[1]
    return pl.pallas_call(
        _layer_body,
        out_shape=jax.ShapeDtypeStruct((NP, 128), jnp.float32),
        grid=(NP // _BR,),
        in_specs=[_row_spec(D), _row_spec(D),
                  _full_spec((D, 128)), _full_spec((1, 128)),
                  _full_spec((D, 128))],
        out_specs=_row_spec(128),
    )(agg, h, Wr, br.reshape(1, 128), Wt)


def _tc_layer_sum(agg, h, other, Wr, br, Wt):
    D = h.shape[1]
    return pl.pallas_call(
        _layer_sum_body,
        out_shape=(jax.ShapeDtypeStruct((NP, 128), jnp.float32),
                   jax.ShapeDtypeStruct((NP, 128), jnp.float32)),
        grid=(NP // _BR,),
        in_specs=[_row_spec(D), _row_spec(D), _row_spec(128),
                  _full_spec((D, 128)), _full_spec((1, 128)),
                  _full_spec((D, 128))],
        out_specs=(_row_spec(128), _row_spec(128)),
    )(agg, h, other, Wr, br.reshape(1, 128), Wt)


def _mlp_body(px, py, pc, Wlin, blin, Wout, bout, out):
    cnt = jnp.maximum(pc[0, :NG] + pc[1, :NG], 1.0)
    mx = (px[0, :NG] + px[1, :NG]) / cnt
    my = (py[0, :NG] + py[1, :NG]) / cnt
    z = jnp.concatenate([mx, my], axis=1)

    def lin(z, i):
        z = _leaky(jnp.dot(z, Wlin[2 * i], preferred_element_type=jnp.float32)
                   + blin[2 * i])
        z = _leaky(jnp.dot(z, Wlin[2 * i + 1], preferred_element_type=jnp.float32)
                   + blin[2 * i + 1])
        return z

    zy = lin(z, 0)
    z = lin(z + zy, 1)
    zy = lin(z + zy, 2)
    z = lin(z + zy, 3)
    out[...] = jnp.dot(z, Wout[...], preferred_element_type=jnp.float32) + bout[...]


def _tc_mlp(px, py, pc, Wlin, blin, Wout, bout):
    Woutp = jnp.zeros((256, 128), jnp.float32).at[:, 0:1].set(Wout)
    boutp = jnp.zeros((1, 128), jnp.float32).at[0, 0].set(bout[0])
    return pl.pallas_call(
        _mlp_body,
        out_shape=jax.ShapeDtypeStruct((NG, 128), jnp.float32),
        in_specs=[pl.BlockSpec((2, NGP, 128), lambda: (0, 0, 0)),
                  pl.BlockSpec((2, NGP, 128), lambda: (0, 0, 0)),
                  pl.BlockSpec((2, NGP, 128), lambda: (0, 0, 0)),
                  pl.BlockSpec((8, 256, 256), lambda: (0, 0, 0)),
                  pl.BlockSpec((8, 1, 256), lambda: (0, 0, 0)),
                  pl.BlockSpec((256, 128), lambda: (0, 0)),
                  pl.BlockSpec((1, 128), lambda: (0, 0))],
        out_specs=pl.BlockSpec((NG, 128), lambda: (0, 0)),
    )(px, py, pc, Wlin, blin.reshape(8, 1, 256), Woutp, boutp)


def kernel(x, edge_index, batch, embed, Wrel0, brel0, Wroot0, Wrel, brel,
           Wroot, Wlin, blin, Wout, bout):
    i32 = jnp.int32
    src3d = jnp.pad(edge_index[0].astype(i32), (0, EPAD - E)).reshape(16, NCH, K)
    dst3d = jnp.pad(edge_index[1].astype(i32), (0, EPAD - E),
                    constant_values=N).reshape(16, NCH, K)
    xp = jnp.pad(x.astype(i32), (0, NP - N))
    x3d = (xp // 2).reshape(32, 4, 80)
    xpar = (xp % 2).astype(jnp.float32).reshape(NP, 1)
    batch3d = jnp.pad(batch.astype(i32), (0, NP - N),
                      constant_values=NG).reshape(32, 4, 80)
    embed2 = embed.reshape(embed.shape[0] // 2, 128)
    Wrel0p = jnp.pad(Wrel0, ((0, 64), (0, 0)))
    Wroot0p = jnp.pad(Wroot0, ((0, 64), (0, 0)))

    h = _tc_fixup(_make_embed_gather()(x3d, embed2), xpar)

    def conv(h, Wr, br, Wt):
        agg = _make_agg(128)(src3d, dst3d, h)
        return _tc_layer(agg, h, Wr, br, Wt)

    def conv_sum(h, other, Wr, br, Wt):
        agg = _make_agg(128)(src3d, dst3d, h)
        return _tc_layer_sum(agg, h, other, Wr, br, Wt)

    h = conv(h, Wrel0p, brel0, Wroot0p)
    xh = conv(h, Wrel[0], brel[0], Wroot[0])

    t = conv(xh, Wrel[1], brel[1], Wroot[1])
    y, s = conv_sum(t, xh, Wrel[2], brel[2], Wroot[2])
    t = conv(s, Wrel[3], brel[3], Wroot[3])
    xh, s = conv_sum(t, y, Wrel[4], brel[4], Wroot[4])
    t = conv(s, Wrel[5], brel[5], Wroot[5])
    y, s = conv_sum(t, xh, Wrel[6], brel[6], Wroot[6])
    t = conv(s, Wrel[7], brel[7], Wroot[7])
    xh, s = conv_sum(t, y, Wrel[8], brel[8], Wroot[8])
    t = conv(s, Wrel[9], brel[9], Wroot[9])
    y = conv(t, Wrel[10], brel[10], Wroot[10])

    px, py, pc = _make_pool()(batch3d, xh, y)
    out = _tc_mlp(px, py, pc, Wlin, blin, Wout, bout)
    return out[:, 0:1]
